# baseline (speedup 1.0000x reference)
import functools

import numpy as np

import jax
import jax.numpy as jnp
from jax import lax
from jax.experimental import pallas as pl
from jax.experimental.pallas import tpu as pltpu
from jax.experimental.pallas import tpu_sc as plsc

N = 100000
NSUB = 16
LANES = 16
CW = 6272
NPAD = NSUB * CW
NVEC = CW // LANES
INT_MAX = 2147483647

_ROT = ((13, 15, 26, 6), (17, 29, 16, 24))


def _np_threefry2x32(k0, k1, x0, x1):
    ks = [np.uint32(k0), np.uint32(k1),
          np.uint32(k0) ^ np.uint32(k1) ^ np.uint32(0x1BD11BDA)]
    x0 = (x0 + ks[0]).astype(np.uint32)
    x1 = (x1 + ks[1]).astype(np.uint32)
    for i in range(5):
        for r in _ROT[i % 2]:
            x0 = (x0 + x1).astype(np.uint32)
            x1 = ((x1 << np.uint32(r)) | (x1 >> np.uint32(32 - r))).astype(np.uint32)
            x1 = x1 ^ x0
        x0 = (x0 + ks[(i + 1) % 3]).astype(np.uint32)
        x1 = (x1 + ks[(i + 2) % 3] + np.uint32(i + 1)).astype(np.uint32)
    return x0, x1


def _derive_keys(seed):
    a, b = _np_threefry2x32(0, seed,
                            np.zeros(2, np.uint32), np.arange(2, dtype=np.uint32))
    return (int(a[0]), int(b[0])), (int(a[1]), int(b[1]))


_K1, _K2 = _derive_keys(42)


def _threefry_draw(key, x1):
    k0, k1 = key
    ks = (jnp.uint32(k0), jnp.uint32(k1),
          jnp.uint32((k0 ^ k1 ^ 0x1BD11BDA) & 0xFFFFFFFF))
    x0 = jnp.zeros_like(x1) + ks[0]
    x1 = x1 + ks[1]
    for i in range(5):
        for r in _ROT[i % 2]:
            x0 = x0 + x1
            x1 = (x1 << r) | (x1 >> (32 - r))
            x1 = x1 ^ x0
        x0 = x0 + ks[(i + 1) % 3]
        x1 = x1 + ks[(i + 2) % 3] + jnp.uint32(i + 1)
    return ((x0 ^ x1) >> 9).astype(jnp.int32)


_BROWS = 56


_BCOLS = _BROWS * 128


def _bits_body(o1_ref, o2_ref, p1_ref, p2_ref):
    step = pl.program_id(0)
    row = lax.broadcasted_iota(jnp.uint32, (_BROWS, 128), 0)
    col = lax.broadcasted_iota(jnp.uint32, (_BROWS, 128), 1)
    idx = (row + jnp.uint32(step * _BROWS)) * 128 + col
    gi = idx.astype(jnp.int32)
    for key, ref in ((_K1, o1_ref), (_K2, o2_ref)):
        ref[...] = jnp.where(gi >= N, -1, _threefry_draw(key, idx))
    p1_ref[...] = jnp.ones((1, _BCOLS), jnp.float32)
    p2_ref[...] = jnp.ones((1, _BCOLS), jnp.float32)


_bits_kernel = pl.pallas_call(
    _bits_body,
    grid=(NPAD // (128 * _BROWS),),
    out_shape=[jax.ShapeDtypeStruct((NPAD // 128, 128), jnp.int32)] * 2
    + [jax.ShapeDtypeStruct((1, N), jnp.float32)] * 2,
    out_specs=[pl.BlockSpec((_BROWS, 128), lambda i: (i, 0))] * 2
    + [pl.BlockSpec((1, _BCOLS), lambda i: (0, i))] * 2,
)


def _partial_argmax(buf, base, excl, lane):
    UNROLL = 8

    def body(j, carry):
        out = []
        for k in range(UNROLL):
            best, bidx = carry[k]
            off = (j * UNROLL + k) * LANES
            v = buf[pl.ds(off, LANES)]
            gidx = lane + (base + off)
            veff = jnp.where(gidx == excl, jnp.int32(-1), v)
            take = veff > best
            out.append((jnp.where(take, veff, best),
                        jnp.where(take, gidx, bidx)))
        return tuple(out)

    init = tuple((jnp.full((LANES,), -1, jnp.int32),
                  jnp.full((LANES,), 0, jnp.int32)) for _ in range(UNROLL))
    chains = lax.fori_loop(0, NVEC // UNROLL, body, init)

    def combine(a, b):
        va, ia = a
        vb, ib = b
        take = (vb > va) | ((vb == va) & (ib < ia))
        return jnp.where(take, vb, va), jnp.where(take, ib, ia)

    level = list(chains)
    while len(level) > 1:
        level = [combine(level[k], level[k + 1])
                 for k in range(0, len(level), 2)]
    best, bidx = level[0]
    vmax = jnp.max(best)
    cand = jnp.where(best == vmax, bidx, INT_MAX)
    return vmax, jnp.min(cand)


def _merge_partials(sh, mbuf, lane):
    pltpu.sync_copy(sh, mbuf)
    zeros = jnp.zeros((LANES,), jnp.int32)
    vals = plsc.load_gather(mbuf, [lane, zeros])
    idxs = plsc.load_gather(mbuf, [lane, zeros + 1])
    gmax = jnp.max(vals)
    cand = jnp.where(vals == gmax, idxs, INT_MAX)
    return jnp.min(cand)


def _publish(stage, sh_row, v, i, lane):
    vrow = jnp.full((LANES,), v, jnp.int32)
    irow = jnp.full((LANES,), i, jnp.int32)
    stage[...] = jnp.where(lane == 0, vrow,
                           jnp.where(lane == 1, irow, jnp.int32(0)))
    pltpu.sync_copy(stage, sh_row)


_SAMPLE_OUT_TYPE = (
    jax.ShapeDtypeStruct((1,), jnp.int32),
    jax.ShapeDtypeStruct((1,), jnp.int32),
)
_SAMPLE_SCRATCH = dict(
    buf1=pltpu.VMEM((CW,), jnp.int32),
    buf2=pltpu.VMEM((CW,), jnp.int32),
    stage=pltpu.VMEM((LANES,), jnp.int32),
    mbuf=pltpu.VMEM((NSUB, LANES), jnp.int32),
    oidx=pltpu.VMEM((LANES,), jnp.int32),
    shp=pltpu.VMEM_SHARED((2 * NSUB, LANES), jnp.int32),
    sem=pltpu.SemaphoreType.DMA,
)


def _sample_body(b1_hbm, b2_hbm, as_hbm, ae_hbm, *,
                 buf1, buf2, stage, mbuf, oidx, shp, sem):
    sh1 = shp.at[pl.ds(0, NSUB)]
    sh2 = shp.at[pl.ds(NSUB, NSUB)]
    sid = lax.axis_index("s")
    lane = lax.iota(jnp.int32, LANES)
    base = sid * CW

    cp2 = pltpu.async_copy(b2_hbm.at[pl.ds(base, CW)], buf2, sem)
    pltpu.sync_copy(b1_hbm.at[pl.ds(base, CW)], buf1)
    v1, i1 = _partial_argmax(buf1, base, jnp.int32(0), lane)
    _publish(stage, sh1.at[sid], v1, i1, lane)
    cp2.wait()

    plsc.subcore_barrier()

    a_start = _merge_partials(sh1, mbuf, lane)
    v2, i2 = _partial_argmax(buf2, base, a_start, lane)
    _publish(stage, sh2.at[sid], v2, i2, lane)

    plsc.subcore_barrier()

    @pl.when(sid == 0)
    def _merge_end():
        a_end = _merge_partials(sh2, mbuf, lane)
        oidx[...] = jnp.where(lane == 0, jnp.full((LANES,), a_start, jnp.int32),
                              jnp.full((LANES,), a_end, jnp.int32))
        pltpu.sync_copy(oidx.at[pl.ds(0, 1)], as_hbm)
        pltpu.sync_copy(oidx.at[pl.ds(8, 1)], ae_hbm)

_sample_kernel = pl.kernel(
    _sample_body,
    out_type=_SAMPLE_OUT_TYPE,
    mesh=plsc.VectorSubcoreMesh(core_axis_name="c", subcore_axis_name="s",
                                num_cores=1, num_subcores=NSUB),
    compiler_params=pltpu.CompilerParams(needs_layout_passes=False),
    scratch_types=_SAMPLE_SCRATCH,
)


def kernel(h, W_emb, b_emb, W1a, b1a, W1b, b1b, W1c, b1c,
           W2a, b2a, W2b, b2b, W2c, b2c):
    b1, b2, ones1, ones2 = _bits_kernel()
    a_start, a_end = _sample_kernel(b1.reshape(NPAD), b2.reshape(NPAD))
    return ones1, a_start, ones2, a_end


# speedup vs baseline: 1.0027x; 1.0027x over previous
import numpy as np

import jax
import jax.numpy as jnp
from jax import lax
from jax.experimental import pallas as pl
from jax.experimental.pallas import tpu as pltpu
from jax.experimental.pallas import tpu_sc as plsc

N = 100000
NSUB = 16
LANES = 16
CW = 6272
NPAD = NSUB * CW
NVEC = CW // LANES
INT_MAX = 2147483647

_ROT = ((13, 15, 26, 6), (17, 29, 16, 24))


def _np_threefry2x32(k0, k1, x0, x1):
    ks = [np.uint32(k0), np.uint32(k1),
          np.uint32(k0) ^ np.uint32(k1) ^ np.uint32(0x1BD11BDA)]
    x0 = (x0 + ks[0]).astype(np.uint32)
    x1 = (x1 + ks[1]).astype(np.uint32)
    for i in range(5):
        for r in _ROT[i % 2]:
            x0 = (x0 + x1).astype(np.uint32)
            x1 = ((x1 << np.uint32(r)) | (x1 >> np.uint32(32 - r))).astype(np.uint32)
            x1 = x1 ^ x0
        x0 = (x0 + ks[(i + 1) % 3]).astype(np.uint32)
        x1 = (x1 + ks[(i + 2) % 3] + np.uint32(i + 1)).astype(np.uint32)
    return x0, x1


def _derive_keys(seed):
    a, b = _np_threefry2x32(0, seed,
                            np.zeros(2, np.uint32), np.arange(2, dtype=np.uint32))
    return (int(a[0]), int(b[0])), (int(a[1]), int(b[1]))


_K1, _K2 = _derive_keys(42)


def _threefry_draw(key, x1):
    k0, k1 = key
    ks = (jnp.uint32(k0), jnp.uint32(k1),
          jnp.uint32((k0 ^ k1 ^ 0x1BD11BDA) & 0xFFFFFFFF))
    x0 = jnp.zeros_like(x1) + ks[0]
    x1 = x1 + ks[1]
    for i in range(5):
        for r in _ROT[i % 2]:
            x0 = x0 + x1
            x1 = (x1 << r) | (x1 >> (32 - r))
            x1 = x1 ^ x0
        x0 = x0 + ks[(i + 1) % 3]
        x1 = x1 + ks[(i + 2) % 3] + jnp.uint32(i + 1)
    return ((x0 ^ x1) >> 9).astype(jnp.int32)


_BROWS = 56


_BCOLS = _BROWS * 128


def _bits_body(o1_ref, o2_ref, p1_ref, p2_ref):
    step = pl.program_id(0)
    row = lax.broadcasted_iota(jnp.uint32, (_BROWS, 128), 0)
    col = lax.broadcasted_iota(jnp.uint32, (_BROWS, 128), 1)
    idx = (row + jnp.uint32(step * _BROWS)) * 128 + col
    gi = idx.astype(jnp.int32)
    for key, ref in ((_K1, o1_ref), (_K2, o2_ref)):
        ref[...] = jnp.where(gi >= N, -1, _threefry_draw(key, idx))
    p1_ref[...] = jnp.ones((1, _BCOLS), jnp.float32)
    p2_ref[...] = jnp.ones((1, _BCOLS), jnp.float32)


_bits_kernel = pl.pallas_call(
    _bits_body,
    grid=(NPAD // (128 * _BROWS),),
    out_shape=[jax.ShapeDtypeStruct((NPAD // 128, 128), jnp.int32)] * 2
    + [jax.ShapeDtypeStruct((1, N), jnp.float32)] * 2,
    out_specs=[pl.BlockSpec((_BROWS, 128), lambda i: (i, 0))] * 2
    + [pl.BlockSpec((1, _BCOLS), lambda i: (0, i))] * 2,
)


def _partial_argmax(buf, base, excl, lane):
    UNROLL = 8

    def body(j, carry):
        out = []
        for k in range(UNROLL):
            best, bidx = carry[k]
            off = (j * UNROLL + k) * LANES
            v = buf[pl.ds(off, LANES)]
            gidx = lane + (base + off)
            veff = jnp.where(gidx == excl, jnp.int32(-1), v)
            take = veff > best
            out.append((jnp.where(take, veff, best),
                        jnp.where(take, gidx, bidx)))
        return tuple(out)

    init = tuple((jnp.full((LANES,), -1, jnp.int32),
                  jnp.full((LANES,), 0, jnp.int32)) for _ in range(UNROLL))
    chains = lax.fori_loop(0, NVEC // UNROLL, body, init)

    def combine(a, b):
        va, ia = a
        vb, ib = b
        take = (vb > va) | ((vb == va) & (ib < ia))
        return jnp.where(take, vb, va), jnp.where(take, ib, ia)

    level = list(chains)
    while len(level) > 1:
        level = [combine(level[k], level[k + 1])
                 for k in range(0, len(level), 2)]
    best, bidx = level[0]
    vmax = jnp.max(best)
    cand = jnp.where(best == vmax, bidx, INT_MAX)
    return vmax, jnp.min(cand)


def _merge_partials(sh, mbuf, lane):
    pltpu.sync_copy(sh, mbuf)
    zeros = jnp.zeros((LANES,), jnp.int32)
    vals = plsc.load_gather(mbuf, [lane, zeros])
    idxs = plsc.load_gather(mbuf, [lane, zeros + 1])
    gmax = jnp.max(vals)
    cand = jnp.where(vals == gmax, idxs, INT_MAX)
    return jnp.min(cand)


def _publish(stage, sh_row, v, i, lane):
    vrow = jnp.full((LANES,), v, jnp.int32)
    irow = jnp.full((LANES,), i, jnp.int32)
    stage[...] = jnp.where(lane == 0, vrow,
                           jnp.where(lane == 1, irow, jnp.int32(0)))
    pltpu.sync_copy(stage, sh_row)


_SAMPLE_OUT_TYPE = (
    jax.ShapeDtypeStruct((1,), jnp.int32),
    jax.ShapeDtypeStruct((1,), jnp.int32),
)
_SAMPLE_SCRATCH = dict(
    buf1=pltpu.VMEM((CW,), jnp.int32),
    buf2=pltpu.VMEM((CW,), jnp.int32),
    stage=pltpu.VMEM((LANES,), jnp.int32),
    mbuf=pltpu.VMEM((NSUB, LANES), jnp.int32),
    oidx=pltpu.VMEM((LANES,), jnp.int32),
    shp=pltpu.VMEM_SHARED((2 * NSUB, LANES), jnp.int32),
    sem=pltpu.SemaphoreType.DMA,
)


def _sample_body(b1_hbm, b2_hbm, as_hbm, ae_hbm, *,
                 buf1, buf2, stage, mbuf, oidx, shp, sem):
    sh1 = shp.at[pl.ds(0, NSUB)]
    sh2 = shp.at[pl.ds(NSUB, NSUB)]
    sid = lax.axis_index("s")
    lane = lax.iota(jnp.int32, LANES)
    base = sid * CW

    cp2 = pltpu.async_copy(b2_hbm.at[pl.ds(base, CW)], buf2, sem)
    pltpu.sync_copy(b1_hbm.at[pl.ds(base, CW)], buf1)
    v1, i1 = _partial_argmax(buf1, base, jnp.int32(0), lane)
    _publish(stage, sh1.at[sid], v1, i1, lane)
    cp2.wait()

    plsc.subcore_barrier()

    a_start = _merge_partials(sh1, mbuf, lane)
    v2, i2 = _partial_argmax(buf2, base, a_start, lane)
    _publish(stage, sh2.at[sid], v2, i2, lane)

    plsc.subcore_barrier()

    @pl.when(sid == 0)
    def _merge_end():
        a_end = _merge_partials(sh2, mbuf, lane)
        oidx[...] = jnp.where(lane == 0, jnp.full((LANES,), a_start, jnp.int32),
                              jnp.full((LANES,), a_end, jnp.int32))
        pltpu.sync_copy(oidx.at[pl.ds(0, 1)], as_hbm)
        pltpu.sync_copy(oidx.at[pl.ds(8, 1)], ae_hbm)

_sample_kernel = pl.kernel(
    _sample_body,
    out_type=_SAMPLE_OUT_TYPE,
    mesh=plsc.VectorSubcoreMesh(core_axis_name="c", subcore_axis_name="s",
                                num_cores=1, num_subcores=NSUB),
    compiler_params=pltpu.CompilerParams(needs_layout_passes=False),
    scratch_types=_SAMPLE_SCRATCH,
)


def kernel(h, W_emb, b_emb, W1a, b1a, W1b, b1b, W1c, b1c,
           W2a, b2a, W2b, b2b, W2c, b2c):
    b1, b2, ones1, ones2 = _bits_kernel()
    a_start, a_end = _sample_kernel(b1.reshape(NPAD), b2.reshape(NPAD))
    return ones1, a_start, ones2, a_end


# speedup vs baseline: 1.0762x; 1.0733x over previous
import numpy as np

import jax
import jax.numpy as jnp
from jax import lax
from jax.experimental import pallas as pl
from jax.experimental.pallas import tpu as pltpu
from jax.experimental.pallas import tpu_sc as plsc

N = 100000
NSUB = 16
LANES = 16
CW = 6272
NPAD = NSUB * CW
NVEC = CW // LANES
INT_MAX = 2147483647

_ROT = ((13, 15, 26, 6), (17, 29, 16, 24))


def _np_threefry2x32(k0, k1, x0, x1):
    ks = [np.uint32(k0), np.uint32(k1),
          np.uint32(k0) ^ np.uint32(k1) ^ np.uint32(0x1BD11BDA)]
    x0 = (x0 + ks[0]).astype(np.uint32)
    x1 = (x1 + ks[1]).astype(np.uint32)
    for i in range(5):
        for r in _ROT[i % 2]:
            x0 = (x0 + x1).astype(np.uint32)
            x1 = ((x1 << np.uint32(r)) | (x1 >> np.uint32(32 - r))).astype(np.uint32)
            x1 = x1 ^ x0
        x0 = (x0 + ks[(i + 1) % 3]).astype(np.uint32)
        x1 = (x1 + ks[(i + 2) % 3] + np.uint32(i + 1)).astype(np.uint32)
    return x0, x1


def _derive_keys(seed):
    a, b = _np_threefry2x32(0, seed,
                            np.zeros(2, np.uint32), np.arange(2, dtype=np.uint32))
    return (int(a[0]), int(b[0])), (int(a[1]), int(b[1]))


_K1, _K2 = _derive_keys(42)


def _threefry_draw(key, x1):
    k0, k1 = key
    ks = (jnp.uint32(k0), jnp.uint32(k1),
          jnp.uint32((k0 ^ k1 ^ 0x1BD11BDA) & 0xFFFFFFFF))
    x0 = jnp.zeros_like(x1) + ks[0]
    x1 = x1 + ks[1]
    for i in range(5):
        for r in _ROT[i % 2]:
            x0 = x0 + x1
            x1 = (x1 << r) | (x1 >> (32 - r))
            x1 = x1 ^ x0
        x0 = x0 + ks[(i + 1) % 3]
        x1 = x1 + ks[(i + 2) % 3] + jnp.uint32(i + 1)
    return ((x0 ^ x1) >> 9).astype(jnp.int32)


_BROWS = 112


_BCOLS = _BROWS * 128


def _bits_body(o1_ref, o2_ref, p1_ref, p2_ref):
    step = pl.program_id(0)
    row = lax.broadcasted_iota(jnp.uint32, (_BROWS, 128), 0)
    col = lax.broadcasted_iota(jnp.uint32, (_BROWS, 128), 1)
    idx = (row + jnp.uint32(step * _BROWS)) * 128 + col
    gi = idx.astype(jnp.int32)
    for key, ref in ((_K1, o1_ref), (_K2, o2_ref)):
        ref[...] = jnp.where(gi >= N, -1, _threefry_draw(key, idx))
    ones = jnp.ones((1, _BCOLS), jnp.float32)
    p1_ref[...] = ones
    p2_ref[...] = ones


_bits_kernel = pl.pallas_call(
    _bits_body,
    grid=(NPAD // (128 * _BROWS),),
    out_shape=[jax.ShapeDtypeStruct((NPAD // 128, 128), jnp.int32)] * 2
    + [jax.ShapeDtypeStruct((1, N), jnp.float32)] * 2,
    out_specs=[pl.BlockSpec((_BROWS, 128), lambda i: (i, 0))] * 2
    + [pl.BlockSpec((1, _BCOLS), lambda i: (0, i))] * 2,
)


def _partial_argmax(buf, base, excl, lane):
    UNROLL = 8

    def body(j, carry):
        out = []
        for k in range(UNROLL):
            best, bidx = carry[k]
            off = (j * UNROLL + k) * LANES
            v = buf[pl.ds(off, LANES)]
            gidx = lane + (base + off)
            veff = jnp.where(gidx == excl, jnp.int32(-1), v)
            take = veff > best
            out.append((jnp.where(take, veff, best),
                        jnp.where(take, gidx, bidx)))
        return tuple(out)

    init = tuple((jnp.full((LANES,), -1, jnp.int32),
                  jnp.full((LANES,), 0, jnp.int32)) for _ in range(UNROLL))
    chains = lax.fori_loop(0, NVEC // UNROLL, body, init)

    def combine(a, b):
        va, ia = a
        vb, ib = b
        take = (vb > va) | ((vb == va) & (ib < ia))
        return jnp.where(take, vb, va), jnp.where(take, ib, ia)

    level = list(chains)
    while len(level) > 1:
        level = [combine(level[k], level[k + 1])
                 for k in range(0, len(level), 2)]
    best, bidx = level[0]
    vmax = jnp.max(best)
    cand = jnp.where(best == vmax, bidx, INT_MAX)
    return vmax, jnp.min(cand)


def _merge_partials(sh, mbuf, lane):
    pltpu.sync_copy(sh, mbuf)
    zeros = jnp.zeros((LANES,), jnp.int32)
    vals = plsc.load_gather(mbuf, [lane, zeros])
    idxs = plsc.load_gather(mbuf, [lane, zeros + 1])
    gmax = jnp.max(vals)
    cand = jnp.where(vals == gmax, idxs, INT_MAX)
    return jnp.min(cand)


def _publish(stage, sh_row, v, i, lane):
    vrow = jnp.full((LANES,), v, jnp.int32)
    irow = jnp.full((LANES,), i, jnp.int32)
    stage[...] = jnp.where(lane == 0, vrow,
                           jnp.where(lane == 1, irow, jnp.int32(0)))
    pltpu.sync_copy(stage, sh_row)


_SAMPLE_OUT_TYPE = (
    jax.ShapeDtypeStruct((1,), jnp.int32),
    jax.ShapeDtypeStruct((1,), jnp.int32),
)
_SAMPLE_SCRATCH = dict(
    buf1=pltpu.VMEM((CW,), jnp.int32),
    buf2=pltpu.VMEM((CW,), jnp.int32),
    stage=pltpu.VMEM((LANES,), jnp.int32),
    mbuf=pltpu.VMEM((NSUB, LANES), jnp.int32),
    oidx=pltpu.VMEM((LANES,), jnp.int32),
    shp=pltpu.VMEM_SHARED((2 * NSUB, LANES), jnp.int32),
    sem=pltpu.SemaphoreType.DMA,
)


def _sample_body(b1_hbm, b2_hbm, as_hbm, ae_hbm, *,
                 buf1, buf2, stage, mbuf, oidx, shp, sem):
    sh1 = shp.at[pl.ds(0, NSUB)]
    sh2 = shp.at[pl.ds(NSUB, NSUB)]
    sid = lax.axis_index("s")
    lane = lax.iota(jnp.int32, LANES)
    base = sid * CW

    cp2 = pltpu.async_copy(b2_hbm.at[pl.ds(base, CW)], buf2, sem)
    pltpu.sync_copy(b1_hbm.at[pl.ds(base, CW)], buf1)
    v1, i1 = _partial_argmax(buf1, base, jnp.int32(0), lane)
    _publish(stage, sh1.at[sid], v1, i1, lane)
    cp2.wait()

    plsc.subcore_barrier()

    a_start = _merge_partials(sh1, mbuf, lane)
    v2, i2 = _partial_argmax(buf2, base, a_start, lane)
    _publish(stage, sh2.at[sid], v2, i2, lane)

    plsc.subcore_barrier()

    @pl.when(sid == 0)
    def _merge_end():
        a_end = _merge_partials(sh2, mbuf, lane)
        oidx[...] = jnp.where(lane == 0, jnp.full((LANES,), a_start, jnp.int32),
                              jnp.full((LANES,), a_end, jnp.int32))
        pltpu.sync_copy(oidx.at[pl.ds(0, 1)], as_hbm)
        pltpu.sync_copy(oidx.at[pl.ds(8, 1)], ae_hbm)

_sample_kernel = pl.kernel(
    _sample_body,
    out_type=_SAMPLE_OUT_TYPE,
    mesh=plsc.VectorSubcoreMesh(core_axis_name="c", subcore_axis_name="s",
                                num_cores=1, num_subcores=NSUB),
    compiler_params=pltpu.CompilerParams(needs_layout_passes=False),
    scratch_types=_SAMPLE_SCRATCH,
)


def kernel(h, W_emb, b_emb, W1a, b1a, W1b, b1b, W1c, b1c,
           W2a, b2a, W2b, b2b, W2c, b2c):
    b1, b2, ones1, ones2 = _bits_kernel()
    a_start, a_end = _sample_kernel(b1.reshape(NPAD), b2.reshape(NPAD))
    return ones1, a_start, ones2, a_end
